# Initial kernel scaffold; baseline (speedup 1.0000x reference)
import jax
import jax.numpy as jnp
from jax.experimental import pallas as pl


def kernel(state_x, state_edge_index, state_edge_type, state_batch, goal_x, goal_edge_index, goal_edge_type, goal_batch, depth, s1_W, s1_root, s1_bias, s2_W, s2_root, s2_bias, g1_W, g1_root, g1_bias, g2_W, g2_root, g2_bias, reg_w1, reg_b1, reg_w2, reg_b2):
    raise NotImplementedError("write your pallas kernel here")


# speedup vs baseline: 3.3599x; 3.3599x over previous
import functools

import jax
import jax.numpy as jnp
from jax import lax
from jax.experimental import pallas as pl
from jax.experimental.pallas import tpu as pltpu
from jax.experimental.pallas import tpu_sc as plsc

N = 10000
E = 160000
D = 128
H = 64
R = 32
B = 64

NC = 2
NS = 16
ET = E // NS
CH = 80
NCHUNK = ET // CH
NROW = N // NS
NRCNT = (N * R) // NS

_MESH = plsc.VectorSubcoreMesh(core_axis_name="c", subcore_axis_name="s")


@functools.partial(
    pl.kernel,
    out_type=jax.ShapeDtypeStruct((NC, NS, 1, ET), jnp.float32),
    mesh=_MESH,
    scratch_types=[
        pltpu.VMEM((NCHUNK, CH), jnp.int32),
        pltpu.VMEM((CH,), jnp.float32),
        pltpu.VMEM((ET,), jnp.float32),
        pltpu.VMEM_SHARED((N * R,), jnp.float32),
    ],
    compiler_params=pltpu.CompilerParams(use_tc_tiling_on_sc=False),
)
def _sc_weights(seg_hbm, w_hbm, seg_v, ones_v, cbuf_v, cnt_sh):
    c = lax.axis_index("c")
    s = lax.axis_index("s")
    pltpu.sync_copy(seg_hbm.at[c, s], seg_v)
    for k in range(CH // 16):
        ones_v[pl.ds(k * 16, 16)] = jnp.ones((16,), jnp.float32)

    def zero_body(i, carry):
        cbuf_v[pl.ds(i * 16, 16)] = jnp.zeros((16,), jnp.float32)
        return carry

    lax.fori_loop(0, ET // 16, zero_body, 0)
    pltpu.sync_copy(cbuf_v, cnt_sh.at[pl.ds(s * NRCNT, ET)])
    pltpu.sync_copy(cbuf_v, cnt_sh.at[pl.ds(s * NRCNT + ET, ET)])
    plsc.subcore_barrier()

    def hist_body(j, carry):
        pltpu.sync_copy(ones_v, cnt_sh.at[seg_v.at[j]], add=True)
        return carry

    lax.fori_loop(0, NCHUNK, hist_body, 0)
    plsc.subcore_barrier()

    def gather_body(j, carry):
        pltpu.sync_copy(cnt_sh.at[seg_v.at[j]], cbuf_v.at[pl.ds(j * CH, CH)])
        return carry

    lax.fori_loop(0, NCHUNK, gather_body, 0)

    def w_body(i, carry):
        cv = cbuf_v[pl.ds(i * 16, 16)]
        cbuf_v[pl.ds(i * 16, 16)] = 1.0 / jnp.maximum(cv, 1.0)
        return carry

    lax.fori_loop(0, ET // 16, w_body, 0)
    pltpu.sync_copy(cbuf_v, w_hbm.at[c, s, 0])


@functools.partial(
    pl.kernel,
    out_type=jax.ShapeDtypeStruct((NC, NS, NROW, H), jnp.float32),
    mesh=_MESH,
    scratch_types=[
        pltpu.VMEM((ET,), jnp.int32),
        pltpu.VMEM((ET,), jnp.float32),
        pltpu.VMEM((NCHUNK, CH), jnp.int32),
        pltpu.VMEM((CH, H), jnp.float32),
        pltpu.VMEM((NROW // 5, H), jnp.float32),
        pltpu.VMEM_SHARED((N, H), jnp.float32),
        pltpu.SemaphoreType.DMA,
    ],
    compiler_params=pltpu.CompilerParams(use_tc_tiling_on_sc=False),
)
def _sc_edge_agg(t_hbm, gidx_hbm, w_hbm, dst_hbm, agg_hbm,
                 gix_v, w_v, dst_v, rows_v, zb_v, agg_sh, sem):
    c = lax.axis_index("c")
    s = lax.axis_index("s")
    pltpu.sync_copy(gidx_hbm.at[c, s, 0], gix_v)
    pltpu.sync_copy(w_hbm.at[c, s, 0], w_v)
    pltpu.sync_copy(dst_hbm.at[c, s], dst_v)

    def zero_body(i, carry):
        for k in range(H // 16):
            zb_v[i, pl.ds(k * 16, 16)] = jnp.zeros((16,), jnp.float32)
        return carry

    lax.fori_loop(0, NROW // 5, zero_body, 0)
    for p in range(5):
        pltpu.sync_copy(zb_v, agg_sh.at[pl.ds(s * NROW + p * (NROW // 5),
                                              NROW // 5)])
    plsc.subcore_barrier()

    def chunk_body(j, carry):
        pltpu.async_copy(t_hbm.at[gix_v.at[pl.ds(j * CH, CH)]], rows_v, sem).wait()
        base = j * CH
        for g in range(CH // 16):
            wv16 = w_v[pl.ds(base + g * 16, 16)]
            for i in range(16):
                e = g * 16 + i
                wb = jnp.full((16,), wv16[i], jnp.float32)
                for k in range(H // 16):
                    sl = pl.ds(k * 16, 16)
                    rows_v[e, sl] = rows_v[e, sl] * wb
        pltpu.sync_copy(rows_v, agg_sh.at[dst_v.at[j]], add=True)
        return carry

    lax.fori_loop(0, NCHUNK, chunk_body, 0)
    plsc.subcore_barrier()
    pltpu.sync_copy(agg_sh.at[pl.ds(s * NROW, NROW)], agg_hbm.at[c, s])


def _tc_transform_body(x_ref, w_ref, rt_ref, t_ref, root_ref):
    r = pl.program_id(1)
    x = x_ref[0]
    t_ref[0, 0] = jnp.dot(x, w_ref[0, 0], preferred_element_type=jnp.float32)

    @pl.when(r == 0)
    def _():
        root_ref[0] = jnp.dot(x, rt_ref[0], preferred_element_type=jnp.float32)


def _transform(x, w, rt, din):
    return pl.pallas_call(
        _tc_transform_body,
        grid=(NC, R),
        in_specs=[
            pl.BlockSpec((1, N, din), lambda s, r: (s, 0, 0)),
            pl.BlockSpec((1, 1, din, H), lambda s, r: (s, r, 0, 0)),
            pl.BlockSpec((1, din, H), lambda s, r: (s, 0, 0)),
        ],
        out_specs=[
            pl.BlockSpec((1, 1, N, H), lambda s, r: (s, r, 0, 0)),
            pl.BlockSpec((1, N, H), lambda s, r: (s, 0, 0)),
        ],
        out_shape=[
            jax.ShapeDtypeStruct((NC, R, N, H), jnp.float32),
            jax.ShapeDtypeStruct((NC, N, H), jnp.float32),
        ],
    )(x, w, rt)


def _tc_layer2_body(agg_ref, root_ref, bias_ref, w_ref, rt_ref, t_ref, root2_ref):
    r = pl.program_id(1)
    h = jnp.maximum(agg_ref[0] + root_ref[0] + bias_ref[0], 0.0)
    t_ref[0, 0] = jnp.dot(h, w_ref[0, 0], preferred_element_type=jnp.float32)

    @pl.when(r == 0)
    def _():
        root2_ref[0] = jnp.dot(h, rt_ref[0], preferred_element_type=jnp.float32)


def _layer2(agg, root, bias, w, rt):
    return pl.pallas_call(
        _tc_layer2_body,
        grid=(NC, R),
        in_specs=[
            pl.BlockSpec((1, N, H), lambda s, r: (s, 0, 0)),
            pl.BlockSpec((1, N, H), lambda s, r: (s, 0, 0)),
            pl.BlockSpec((1, 1, H), lambda s, r: (s, 0, 0)),
            pl.BlockSpec((1, 1, H, H), lambda s, r: (s, r, 0, 0)),
            pl.BlockSpec((1, H, H), lambda s, r: (s, 0, 0)),
        ],
        out_specs=[
            pl.BlockSpec((1, 1, N, H), lambda s, r: (s, r, 0, 0)),
            pl.BlockSpec((1, N, H), lambda s, r: (s, 0, 0)),
        ],
        out_shape=[
            jax.ShapeDtypeStruct((NC, R, N, H), jnp.float32),
            jax.ShapeDtypeStruct((NC, N, H), jnp.float32),
        ],
    )(agg, root, bias, w, rt)


def _tc_final_body(agg_ref, root_ref, bias_ref, batch_ref, depth_ref,
                   w1a_ref, w1b_ref, b1_ref, w2_ref, b2_ref, out_ref):
    embs = []
    for s in range(2):
        h = jnp.maximum(agg_ref[s] + root_ref[s] + bias_ref[s], 0.0)
        bids = batch_ref[pl.ds(s, 1)]
        onehot = (lax.broadcasted_iota(jnp.int32, (B, N), 0) == bids
                  ).astype(jnp.float32)
        ssum = jnp.dot(onehot, h, preferred_element_type=jnp.float32)
        cnt = jnp.sum(onehot, axis=1)
        embs.append(ssum / jnp.maximum(cnt, 1.0)[:, None])
    emb = jnp.concatenate(embs, axis=1)
    d = depth_ref[...]
    dm = jnp.mean(d)
    dstd = jnp.sqrt(jnp.mean((d - dm) ** 2))
    dn = (d - dm) / (dstd + 1e-6)
    z1 = (jnp.dot(emb, w1a_ref[...], preferred_element_type=jnp.float32)
          + dn * w1b_ref[...] + b1_ref[...])
    h1 = jnp.maximum(z1, 0.0)
    out_ref[...] = (jnp.dot(h1, w2_ref[...], preferred_element_type=jnp.float32)
                    + b2_ref[...])


def _final(agg, root, bias, batch, depth, w1a, w1b, b1, w2, b2):
    return pl.pallas_call(
        _tc_final_body,
        out_shape=jax.ShapeDtypeStruct((B, 1), jnp.float32),
    )(agg, root, bias, batch, depth, w1a, w1b, b1, w2, b2)


def kernel(state_x, state_edge_index, state_edge_type, state_batch,
           goal_x, goal_edge_index, goal_edge_type, goal_batch, depth,
           s1_W, s1_root, s1_bias, s2_W, s2_root, s2_bias,
           g1_W, g1_root, g1_bias, g2_W, g2_root, g2_bias,
           reg_w1, reg_b1, reg_w2, reg_b2):
    f32 = jnp.float32
    x = jnp.stack([state_x, goal_x])
    w1 = jnp.stack([s1_W, g1_W])
    rt1 = jnp.stack([s1_root, g1_root])
    b1 = jnp.stack([s1_bias, g1_bias]).reshape(NC, 1, H)
    w2 = jnp.stack([s2_W, g2_W])
    rt2 = jnp.stack([s2_root, g2_root])
    b2 = jnp.stack([s2_bias, g2_bias]).reshape(NC, 1, H)

    src = jnp.stack([state_edge_index[0], goal_edge_index[0]])
    dst = jnp.stack([state_edge_index[1], goal_edge_index[1]])
    et = jnp.stack([state_edge_type, goal_edge_type])
    seg = (dst * R + et).reshape(NC, NS, NCHUNK, CH)
    enc_off = jnp.arange(NC, dtype=jnp.int32)[:, None] * (R * N)
    gidx = (et * N + src + enc_off).reshape(NC, NS, 1, ET)
    dst4 = dst.reshape(NC, NS, NCHUNK, CH)
    batch = jnp.stack([state_batch, goal_batch])

    w_edge = _sc_weights(seg)

    t1, root1 = _transform(x, w1, rt1, D)
    agg1 = _sc_edge_agg(t1.reshape(NC * R * N, H), gidx, w_edge, dst4)
    t2, root2 = _layer2(agg1.reshape(NC, N, H), root1, b1, w2, rt2)
    agg2 = _sc_edge_agg(t2.reshape(NC * R * N, H), gidx, w_edge, dst4)
    agg2 = agg2.reshape(NC, N, H)

    pred = _final(agg2, root2, b2, batch, depth.reshape(B, 1),
                  reg_w1[:2 * H], reg_w1[2 * H:], reg_b1.reshape(1, H),
                  reg_w2, reg_b2.reshape(1, 1))
    return pred[:, 0]


# speedup vs baseline: 4.8049x; 1.4301x over previous
import functools

import jax
import jax.numpy as jnp
from jax import lax
from jax.experimental import pallas as pl
from jax.experimental.pallas import tpu as pltpu
from jax.experimental.pallas import tpu_sc as plsc

N = 10000
E = 160000
D = 128
H = 64
R = 32
B = 64

NC = 2
NS = 16
NW = NC * NS
ET = E // NS
CH = 80
NCHUNK = ET // CH
NROW = N // NS
NRCNT = (N * R) // NS
NP = N // 2
EP = 163840
ET2 = EP // NW
NCH2 = ET2 // CH

_MESH = plsc.VectorSubcoreMesh(core_axis_name="c", subcore_axis_name="s")
_SC_PARAMS = pltpu.CompilerParams(use_tc_tiling_on_sc=False)


@functools.partial(
    pl.kernel,
    out_type=jax.ShapeDtypeStruct((NC, NS, 1, ET), jnp.float32),
    mesh=_MESH,
    scratch_types=[
        pltpu.VMEM((NCHUNK, CH), jnp.int32),
        pltpu.VMEM((CH,), jnp.float32),
        pltpu.VMEM((ET,), jnp.float32),
        pltpu.VMEM_SHARED((N * R,), jnp.float32),
    ],
    compiler_params=_SC_PARAMS,
)
def _sc_weights(seg_hbm, w_hbm, seg_v, ones_v, cbuf_v, cnt_sh):
    c = lax.axis_index("c")
    s = lax.axis_index("s")
    pltpu.sync_copy(seg_hbm.at[c, s], seg_v)
    for k in range(CH // 16):
        ones_v[pl.ds(k * 16, 16)] = jnp.ones((16,), jnp.float32)

    def zero_body(i, carry):
        cbuf_v[pl.ds(i * 16, 16)] = jnp.zeros((16,), jnp.float32)
        return carry

    lax.fori_loop(0, ET // 16, zero_body, 0)
    pltpu.sync_copy(cbuf_v, cnt_sh.at[pl.ds(s * NRCNT, ET)])
    pltpu.sync_copy(cbuf_v, cnt_sh.at[pl.ds(s * NRCNT + ET, ET)])
    plsc.subcore_barrier()

    def hist_body(j, carry):
        pltpu.sync_copy(ones_v, cnt_sh.at[seg_v.at[j]], add=True)
        return carry

    lax.fori_loop(0, NCHUNK, hist_body, 0)
    plsc.subcore_barrier()

    def gather_body(j, carry):
        pltpu.sync_copy(cnt_sh.at[seg_v.at[j]], cbuf_v.at[pl.ds(j * CH, CH)])
        return carry

    lax.fori_loop(0, NCHUNK, gather_body, 0)

    def w_body(i, carry):
        cv = cbuf_v[pl.ds(i * 16, 16)]
        cbuf_v[pl.ds(i * 16, 16)] = 1.0 / jnp.maximum(cv, 1.0)
        return carry

    lax.fori_loop(0, ET // 16, w_body, 0)
    pltpu.sync_copy(cbuf_v, w_hbm.at[c, s, 0])


@functools.partial(
    pl.kernel,
    out_type=jax.ShapeDtypeStruct((NC, NS, NROW, H), jnp.float32),
    mesh=_MESH,
    scratch_types=[
        pltpu.VMEM((ET2,), jnp.int32),
        pltpu.VMEM((ET2,), jnp.float32),
        pltpu.VMEM((NCH2, CH), jnp.int32),
        pltpu.VMEM((CH, H), jnp.float32),
        pltpu.VMEM((CH, H), jnp.float32),
        pltpu.VMEM((CH, H), jnp.float32),
        pltpu.VMEM((CH, H), jnp.float32),
        pltpu.VMEM((NROW // 5, H), jnp.float32),
        pltpu.VMEM_SHARED((N, H), jnp.float32),
        pltpu.SemaphoreType.DMA,
        pltpu.SemaphoreType.DMA,
        pltpu.SemaphoreType.DMA,
        pltpu.SemaphoreType.DMA,
    ],
    compiler_params=_SC_PARAMS,
)
def _sc_edge_agg(t_hbm, gidx_hbm, w_hbm, dst_hbm, agg_hbm,
                 gix_v, w_v, dst_v, g0_v, g1_v, s0_v, s1_v, zb_v, agg_sh,
                 semg0, semg1, sems0, sems1):
    c = lax.axis_index("c")
    s = lax.axis_index("s")
    widx = s * NC + c
    pltpu.sync_copy(gidx_hbm.at[widx, 0], gix_v)
    pltpu.sync_copy(w_hbm.at[widx, 0], w_v)
    pltpu.sync_copy(dst_hbm.at[widx], dst_v)

    def zero_body(i, carry):
        for k in range(H // 16):
            z = jnp.zeros((16,), jnp.float32)
            zb_v[i, pl.ds(k * 16, 16)] = z
        return carry

    lax.fori_loop(0, NROW // 5, zero_body, 0)

    def zero_sbuf(i, carry):
        for k in range(H // 16):
            z = jnp.zeros((16,), jnp.float32)
            s0_v[i, pl.ds(k * 16, 16)] = z
            s1_v[i, pl.ds(k * 16, 16)] = z
        return carry

    lax.fori_loop(0, CH, zero_sbuf, 0)
    for p in range(5):
        pltpu.sync_copy(zb_v, agg_sh.at[pl.ds(s * NROW + p * (NROW // 5),
                                              NROW // 5)])
    plsc.subcore_barrier()

    def _gather(j, buf, sem):
        return pltpu.async_copy(t_hbm.at[gix_v.at[pl.ds(j * CH, CH)]], buf, sem)

    def _scale(j_base, gbuf, sbuf):
        for g in range(CH // 16):
            wv16 = w_v[pl.ds(j_base + g * 16, 16)]
            for i in range(16):
                e = g * 16 + i
                wb = jnp.full((16,), wv16[i], jnp.float32)
                for k in range(H // 16):
                    sl = pl.ds(k * 16, 16)
                    sbuf[e, sl] = gbuf[e, sl] * wb

    def _scatter(j, buf, sem):
        return pltpu.async_copy(buf, agg_sh.at[dst_v.at[j]], sem, add=True)

    _gather(0, g0_v, semg0)
    _gather(1, g1_v, semg1)
    _scatter(0, s0_v, sems0)
    _scatter(0, s1_v, sems1)

    def chunk_pair(m, carry):
        j0 = 2 * m
        j1 = j0 + 1
        pltpu.make_async_copy(t_hbm, g0_v, semg0).wait()
        pltpu.make_async_copy(s0_v, agg_sh.at[dst_v.at[0]], sems0).wait()
        _scale(j0 * CH, g0_v, s0_v)
        _scatter(j0, s0_v, sems0)
        _gather(jnp.minimum(j0 + 2, NCH2 - 1), g0_v, semg0)
        pltpu.make_async_copy(t_hbm, g1_v, semg1).wait()
        pltpu.make_async_copy(s1_v, agg_sh.at[dst_v.at[0]], sems1).wait()
        _scale(j1 * CH, g1_v, s1_v)
        _scatter(j1, s1_v, sems1)
        _gather(jnp.minimum(j1 + 2, NCH2 - 1), g1_v, semg1)
        return carry

    lax.fori_loop(0, NCH2 // 2, chunk_pair, 0)
    pltpu.make_async_copy(t_hbm, g0_v, semg0).wait()
    pltpu.make_async_copy(t_hbm, g1_v, semg1).wait()
    pltpu.make_async_copy(s0_v, agg_sh.at[dst_v.at[0]], sems0).wait()
    pltpu.make_async_copy(s1_v, agg_sh.at[dst_v.at[0]], sems1).wait()
    plsc.subcore_barrier()
    pltpu.sync_copy(agg_sh.at[pl.ds(s * NROW, NROW)], agg_hbm.at[c, s])


def _tc_transform_body(x_ref, w_ref, rt_ref, t_ref, root_ref):
    r = pl.program_id(0)
    x = x_ref[...]
    w = w_ref[0]
    t_ref[0, :, 0:H] = jnp.dot(x[0:NP], w, preferred_element_type=jnp.float32)
    t_ref[0, :, H:2 * H] = jnp.dot(x[NP:N], w,
                                   preferred_element_type=jnp.float32)

    @pl.when(r == 0)
    def _():
        root_ref[...] = jnp.dot(x, rt_ref[...],
                                preferred_element_type=jnp.float32)


def _transform(x, w, rt, din):
    return pl.pallas_call(
        _tc_transform_body,
        grid=(R,),
        in_specs=[
            pl.BlockSpec((N, din), lambda r: (0, 0)),
            pl.BlockSpec((1, din, H), lambda r: (r, 0, 0)),
            pl.BlockSpec((din, H), lambda r: (0, 0)),
        ],
        out_specs=[
            pl.BlockSpec((1, NP, 2 * H), lambda r: (r, 0, 0)),
            pl.BlockSpec((N, H), lambda r: (0, 0)),
        ],
        out_shape=[
            jax.ShapeDtypeStruct((R, NP, 2 * H), jnp.float32),
            jax.ShapeDtypeStruct((N, H), jnp.float32),
        ],
    )(x, w, rt)


def _tc_layer2_body(agg_ref, root_ref, bias_ref, w_ref, rt_ref, t_ref,
                    root2_ref):
    r = pl.program_id(0)
    h = jnp.maximum(agg_ref[0] + agg_ref[1] + root_ref[...] + bias_ref[...],
                    0.0)
    w = w_ref[0]
    t_ref[0, :, 0:H] = jnp.dot(h[0:NP], w, preferred_element_type=jnp.float32)
    t_ref[0, :, H:2 * H] = jnp.dot(h[NP:N], w,
                                   preferred_element_type=jnp.float32)

    @pl.when(r == 0)
    def _():
        root2_ref[...] = jnp.dot(h, rt_ref[...],
                                 preferred_element_type=jnp.float32)


def _layer2(agg, root, bias, w, rt):
    return pl.pallas_call(
        _tc_layer2_body,
        grid=(R,),
        in_specs=[
            pl.BlockSpec((NC, N, H), lambda r: (0, 0, 0)),
            pl.BlockSpec((N, H), lambda r: (0, 0)),
            pl.BlockSpec((1, H), lambda r: (0, 0)),
            pl.BlockSpec((1, H, H), lambda r: (r, 0, 0)),
            pl.BlockSpec((H, H), lambda r: (0, 0)),
        ],
        out_specs=[
            pl.BlockSpec((1, NP, 2 * H), lambda r: (r, 0, 0)),
            pl.BlockSpec((N, H), lambda r: (0, 0)),
        ],
        out_shape=[
            jax.ShapeDtypeStruct((R, NP, 2 * H), jnp.float32),
            jax.ShapeDtypeStruct((N, H), jnp.float32),
        ],
    )(agg, root, bias, w, rt)


def _tc_final_body(as_ref, ag_ref, rs_ref, rg_ref, bs_ref, bg_ref,
                   batch_ref, depth_ref, w1a_ref, w1b_ref, b1_ref,
                   w2_ref, b2_ref, out_ref):
    embs = []
    for s, (a_ref, r_ref, b_ref) in enumerate(
            [(as_ref, rs_ref, bs_ref), (ag_ref, rg_ref, bg_ref)]):
        h = jnp.maximum(a_ref[0] + a_ref[1] + r_ref[...] + b_ref[...], 0.0)
        bids = batch_ref[pl.ds(s, 1)]
        onehot = (lax.broadcasted_iota(jnp.int32, (B, N), 0) == bids
                  ).astype(jnp.float32)
        ssum = jnp.dot(onehot, h, preferred_element_type=jnp.float32)
        cnt = jnp.sum(onehot, axis=1)
        embs.append(ssum / jnp.maximum(cnt, 1.0)[:, None])
    emb = jnp.concatenate(embs, axis=1)
    d = depth_ref[...]
    dm = jnp.mean(d)
    dstd = jnp.sqrt(jnp.mean((d - dm) ** 2))
    dn = (d - dm) / (dstd + 1e-6)
    z1 = (jnp.dot(emb, w1a_ref[...], preferred_element_type=jnp.float32)
          + dn * w1b_ref[...] + b1_ref[...])
    h1 = jnp.maximum(z1, 0.0)
    out_ref[...] = (jnp.dot(h1, w2_ref[...], preferred_element_type=jnp.float32)
                    + b2_ref[...])


def _final(agg_s, agg_g, root_s, root_g, bias_s, bias_g, batch, depth,
           w1a, w1b, b1, w2, b2):
    return pl.pallas_call(
        _tc_final_body,
        out_shape=jax.ShapeDtypeStruct((B, 1), jnp.float32),
    )(agg_s, agg_g, root_s, root_g, bias_s, bias_g, batch, depth,
      w1a, w1b, b1, w2, b2)


def _edge_tables(edge_index, edge_type, w_flat):
    src = edge_index[0]
    dst = edge_index[1]
    et = edge_type
    pad = EP - E
    zpad = jnp.zeros((pad,), jnp.int32)
    srcp = jnp.concatenate([src, zpad])
    dstp = jnp.concatenate([dst, zpad])
    etp = jnp.concatenate([et, zpad])
    wp = jnp.concatenate([w_flat, jnp.zeros((pad,), jnp.float32)])
    gidx = (2 * (etp * NP + jnp.remainder(srcp, NP)) + srcp // NP)
    return (gidx.reshape(NW, 1, ET2), wp.reshape(NW, 1, ET2),
            dstp.reshape(NW, NCH2, CH))


def kernel(state_x, state_edge_index, state_edge_type, state_batch,
           goal_x, goal_edge_index, goal_edge_type, goal_batch, depth,
           s1_W, s1_root, s1_bias, s2_W, s2_root, s2_bias,
           g1_W, g1_root, g1_bias, g2_W, g2_root, g2_bias,
           reg_w1, reg_b1, reg_w2, reg_b2):
    dst2 = jnp.stack([state_edge_index[1], goal_edge_index[1]])
    et2 = jnp.stack([state_edge_type, goal_edge_type])
    seg = (dst2 * R + et2).reshape(NC, NS, NCHUNK, CH)
    w_edge = _sc_weights(seg)
    w2flat = w_edge.reshape(NC, E)

    gidx_s, w_s, dst_s = _edge_tables(state_edge_index, state_edge_type,
                                      w2flat[0])
    gidx_g, w_g, dst_g = _edge_tables(goal_edge_index, goal_edge_type,
                                      w2flat[1])

    t1s, root1s = _transform(state_x, s1_W, s1_root, D)
    t1g, root1g = _transform(goal_x, g1_W, g1_root, D)
    agg1s = _sc_edge_agg(t1s.reshape(R * N, H), gidx_s, w_s, dst_s)
    agg1g = _sc_edge_agg(t1g.reshape(R * N, H), gidx_g, w_g, dst_g)

    t2s, root2s = _layer2(agg1s.reshape(NC, N, H), root1s,
                          s1_bias.reshape(1, H), s2_W, s2_root)
    t2g, root2g = _layer2(agg1g.reshape(NC, N, H), root1g,
                          g1_bias.reshape(1, H), g2_W, g2_root)
    agg2s = _sc_edge_agg(t2s.reshape(R * N, H), gidx_s, w_s, dst_s)
    agg2g = _sc_edge_agg(t2g.reshape(R * N, H), gidx_g, w_g, dst_g)

    batch = jnp.stack([state_batch, goal_batch])
    pred = _final(agg2s.reshape(NC, N, H), agg2g.reshape(NC, N, H),
                  root2s, root2g, s2_bias.reshape(1, H),
                  g2_bias.reshape(1, H), batch, depth.reshape(B, 1),
                  reg_w1[:2 * H], reg_w1[2 * H:], reg_b1.reshape(1, H),
                  reg_w2, reg_b2.reshape(1, 1))
    return pred[:, 0]


# speedup vs baseline: 7.2846x; 1.5161x over previous
import functools

import jax
import jax.numpy as jnp
from jax import lax
from jax.experimental import pallas as pl
from jax.experimental.pallas import tpu as pltpu
from jax.experimental.pallas import tpu_sc as plsc

N = 10000
E = 160000
D = 128
H = 64
R = 32
B = 64

NC = 2
NS = 16
ET = E // NS
CH = 80
NCHUNK = ET // CH
NROW = N // NS
NRCNT = (N * R) // NS
NP = N // 2

_MESH = plsc.VectorSubcoreMesh(core_axis_name="c", subcore_axis_name="s")
_SC_PARAMS = pltpu.CompilerParams(use_tc_tiling_on_sc=False)

_EDGE_AGG_SCRATCH = [
    pltpu.VMEM((ET,), jnp.int32),
    pltpu.VMEM((ET,), jnp.float32),
    pltpu.VMEM((NCHUNK, CH), jnp.int32),
    pltpu.VMEM((CH, H), jnp.float32),
    pltpu.VMEM((CH, H), jnp.float32),
    pltpu.VMEM((CH, H), jnp.float32),
    pltpu.VMEM((CH, H), jnp.float32),
    pltpu.VMEM((NROW // 5, H), jnp.float32),
    pltpu.VMEM_SHARED((N, H), jnp.float32),
    pltpu.SemaphoreType.DMA,
    pltpu.SemaphoreType.DMA,
    pltpu.SemaphoreType.DMA,
    pltpu.SemaphoreType.DMA,
]


def _zero_fill(ref, nrows):
    def body(i, carry):
        for k in range(H // 16):
            ref[i, pl.ds(k * 16, 16)] = jnp.zeros((16,), jnp.float32)
        return carry

    lax.fori_loop(0, nrows, body, 0)


def _edge_agg_pipeline(t_hbm, agg_hbm, gix_v, w_v, dst_v,
                       g0_v, g1_v, s0_v, s1_v, zb_v, agg_sh,
                       semg0, semg1, sems0, sems1, c, s):
    _zero_fill(zb_v, NROW // 5)
    _zero_fill(s0_v, CH)
    _zero_fill(s1_v, CH)
    for p in range(5):
        pltpu.sync_copy(zb_v, agg_sh.at[pl.ds(s * NROW + p * (NROW // 5),
                                              NROW // 5)])
    plsc.subcore_barrier()

    def _gather(j, buf, sem):
        return pltpu.async_copy(t_hbm.at[gix_v.at[pl.ds(j * CH, CH)]], buf, sem)

    def _scale(j_base, gbuf, sbuf):
        for g in range(CH // 16):
            wv16 = w_v[pl.ds(j_base + g * 16, 16)]
            for i in range(16):
                e = g * 16 + i
                wb = jnp.full((16,), wv16[i], jnp.float32)
                for k in range(H // 16):
                    sl = pl.ds(k * 16, 16)
                    sbuf[e, sl] = gbuf[e, sl] * wb

    def _scatter(j, buf, sem):
        return pltpu.async_copy(buf, agg_sh.at[dst_v.at[j]], sem, add=True)

    _gather(0, g0_v, semg0)
    _gather(1, g1_v, semg1)
    _scatter(0, s0_v, sems0)
    _scatter(0, s1_v, sems1)

    def chunk_pair(m, carry):
        j0 = 2 * m
        j1 = j0 + 1
        pltpu.make_async_copy(t_hbm, g0_v, semg0).wait()
        pltpu.make_async_copy(s0_v, agg_sh.at[dst_v.at[0]], sems0).wait()
        _scale(j0 * CH, g0_v, s0_v)
        _scatter(j0, s0_v, sems0)
        _gather(j0 + 2, g0_v, semg0)
        pltpu.make_async_copy(t_hbm, g1_v, semg1).wait()
        pltpu.make_async_copy(s1_v, agg_sh.at[dst_v.at[0]], sems1).wait()
        _scale(j1 * CH, g1_v, s1_v)
        _scatter(j1, s1_v, sems1)
        _gather(jnp.minimum(j1 + 2, NCHUNK - 1), g1_v, semg1)
        return carry

    lax.fori_loop(0, (NCHUNK - 1) // 2, chunk_pair, 0)
    pltpu.make_async_copy(t_hbm, g0_v, semg0).wait()
    pltpu.make_async_copy(s0_v, agg_sh.at[dst_v.at[0]], sems0).wait()
    _scale((NCHUNK - 1) * CH, g0_v, s0_v)
    _scatter(NCHUNK - 1, s0_v, sems0)
    pltpu.make_async_copy(t_hbm, g1_v, semg1).wait()
    pltpu.make_async_copy(s0_v, agg_sh.at[dst_v.at[0]], sems0).wait()
    pltpu.make_async_copy(s1_v, agg_sh.at[dst_v.at[0]], sems1).wait()
    plsc.subcore_barrier()
    pltpu.sync_copy(agg_sh.at[pl.ds(s * NROW, NROW)], agg_hbm.at[c, s])


@functools.partial(
    pl.kernel,
    out_type=[
        jax.ShapeDtypeStruct((NC, NS, 1, ET), jnp.float32),
        jax.ShapeDtypeStruct((NC, NS, NROW, H), jnp.float32),
    ],
    mesh=_MESH,
    scratch_types=[
        pltpu.VMEM((NCHUNK, CH), jnp.int32),
        pltpu.VMEM((CH,), jnp.float32),
        pltpu.VMEM_SHARED((N * R,), jnp.float32),
    ] + _EDGE_AGG_SCRATCH,
    compiler_params=_SC_PARAMS,
)
def _sc_weights_agg1(seg_hbm, t_hbm, gidx_hbm, dst_hbm, w_hbm, agg_hbm,
                     seg_v, ones_v, cnt_sh,
                     gix_v, w_v, dst_v, g0_v, g1_v, s0_v, s1_v, zb_v, agg_sh,
                     semg0, semg1, sems0, sems1):
    c = lax.axis_index("c")
    s = lax.axis_index("s")
    pltpu.sync_copy(seg_hbm.at[c, s], seg_v)
    pltpu.sync_copy(gidx_hbm.at[c, s, 0], gix_v)
    pltpu.sync_copy(dst_hbm.at[c, s], dst_v)
    for k in range(CH // 16):
        ones_v[pl.ds(k * 16, 16)] = jnp.ones((16,), jnp.float32)

    def zero_body(i, carry):
        w_v[pl.ds(i * 16, 16)] = jnp.zeros((16,), jnp.float32)
        return carry

    lax.fori_loop(0, ET // 16, zero_body, 0)
    pltpu.sync_copy(w_v, cnt_sh.at[pl.ds(s * NRCNT, ET)])
    pltpu.sync_copy(w_v, cnt_sh.at[pl.ds(s * NRCNT + ET, ET)])
    plsc.subcore_barrier()

    def hist_body(j, carry):
        pltpu.sync_copy(ones_v, cnt_sh.at[seg_v.at[j]], add=True)
        return carry

    lax.fori_loop(0, NCHUNK, hist_body, 0)
    plsc.subcore_barrier()

    def gather_body(j, carry):
        pltpu.sync_copy(cnt_sh.at[seg_v.at[j]], w_v.at[pl.ds(j * CH, CH)])
        return carry

    lax.fori_loop(0, NCHUNK, gather_body, 0)

    def w_body(i, carry):
        cv = w_v[pl.ds(i * 16, 16)]
        w_v[pl.ds(i * 16, 16)] = 1.0 / jnp.maximum(cv, 1.0)
        return carry

    lax.fori_loop(0, ET // 16, w_body, 0)
    pltpu.sync_copy(w_v, w_hbm.at[c, s, 0])

    _edge_agg_pipeline(t_hbm, agg_hbm, gix_v, w_v, dst_v,
                       g0_v, g1_v, s0_v, s1_v, zb_v, agg_sh,
                       semg0, semg1, sems0, sems1, c, s)


@functools.partial(
    pl.kernel,
    out_type=jax.ShapeDtypeStruct((NC, NS, NROW, H), jnp.float32),
    mesh=_MESH,
    scratch_types=_EDGE_AGG_SCRATCH,
    compiler_params=_SC_PARAMS,
)
def _sc_edge_agg(t_hbm, gidx_hbm, w_hbm, dst_hbm, agg_hbm,
                 gix_v, w_v, dst_v, g0_v, g1_v, s0_v, s1_v, zb_v, agg_sh,
                 semg0, semg1, sems0, sems1):
    c = lax.axis_index("c")
    s = lax.axis_index("s")
    pltpu.sync_copy(gidx_hbm.at[c, s, 0], gix_v)
    pltpu.sync_copy(w_hbm.at[c, s, 0], w_v)
    pltpu.sync_copy(dst_hbm.at[c, s], dst_v)
    _edge_agg_pipeline(t_hbm, agg_hbm, gix_v, w_v, dst_v,
                       g0_v, g1_v, s0_v, s1_v, zb_v, agg_sh,
                       semg0, semg1, sems0, sems1, c, s)


def _tc_transform_body(x_ref, w_ref, rt_ref, t_ref, root_ref):
    r = pl.program_id(1)
    x = x_ref[0]
    w = w_ref[0, 0]
    t_ref[0, 0, :, 0:H] = jnp.dot(x[0:NP], w, preferred_element_type=jnp.float32)
    t_ref[0, 0, :, H:2 * H] = jnp.dot(x[NP:N], w,
                                      preferred_element_type=jnp.float32)

    @pl.when(r == 0)
    def _():
        root_ref[0] = jnp.dot(x, rt_ref[0], preferred_element_type=jnp.float32)


def _transform(x, w, rt, din):
    return pl.pallas_call(
        _tc_transform_body,
        grid=(NC, R),
        in_specs=[
            pl.BlockSpec((1, N, din), lambda s, r: (s, 0, 0)),
            pl.BlockSpec((1, 1, din, H), lambda s, r: (s, r, 0, 0)),
            pl.BlockSpec((1, din, H), lambda s, r: (s, 0, 0)),
        ],
        out_specs=[
            pl.BlockSpec((1, 1, NP, 2 * H), lambda s, r: (s, r, 0, 0)),
            pl.BlockSpec((1, N, H), lambda s, r: (s, 0, 0)),
        ],
        out_shape=[
            jax.ShapeDtypeStruct((NC, R, NP, 2 * H), jnp.float32),
            jax.ShapeDtypeStruct((NC, N, H), jnp.float32),
        ],
    )(x, w, rt)


def _tc_layer2_body(agg_ref, root_ref, bias_ref, w_ref, rt_ref, t_ref, root2_ref):
    r = pl.program_id(1)
    h = jnp.maximum(agg_ref[0] + root_ref[0] + bias_ref[0], 0.0)
    w = w_ref[0, 0]
    t_ref[0, 0, :, 0:H] = jnp.dot(h[0:NP], w, preferred_element_type=jnp.float32)
    t_ref[0, 0, :, H:2 * H] = jnp.dot(h[NP:N], w,
                                      preferred_element_type=jnp.float32)

    @pl.when(r == 0)
    def _():
        root2_ref[0] = jnp.dot(h, rt_ref[0], preferred_element_type=jnp.float32)


def _layer2(agg, root, bias, w, rt):
    return pl.pallas_call(
        _tc_layer2_body,
        grid=(NC, R),
        in_specs=[
            pl.BlockSpec((1, N, H), lambda s, r: (s, 0, 0)),
            pl.BlockSpec((1, N, H), lambda s, r: (s, 0, 0)),
            pl.BlockSpec((1, 1, H), lambda s, r: (s, 0, 0)),
            pl.BlockSpec((1, 1, H, H), lambda s, r: (s, r, 0, 0)),
            pl.BlockSpec((1, H, H), lambda s, r: (s, 0, 0)),
        ],
        out_specs=[
            pl.BlockSpec((1, 1, NP, 2 * H), lambda s, r: (s, r, 0, 0)),
            pl.BlockSpec((1, N, H), lambda s, r: (s, 0, 0)),
        ],
        out_shape=[
            jax.ShapeDtypeStruct((NC, R, NP, 2 * H), jnp.float32),
            jax.ShapeDtypeStruct((NC, N, H), jnp.float32),
        ],
    )(agg, root, bias, w, rt)


def _tc_final_body(agg_ref, root_ref, bias_ref, batch_ref, depth_ref,
                   w1a_ref, w1b_ref, b1_ref, w2_ref, b2_ref, out_ref):
    embs = []
    for s in range(2):
        h = jnp.maximum(agg_ref[s] + root_ref[s] + bias_ref[s], 0.0)
        bids = batch_ref[pl.ds(s, 1)]
        onehot = (lax.broadcasted_iota(jnp.int32, (B, N), 0) == bids
                  ).astype(jnp.float32)
        ssum = jnp.dot(onehot, h, preferred_element_type=jnp.float32)
        cnt = jnp.sum(onehot, axis=1)
        embs.append(ssum / jnp.maximum(cnt, 1.0)[:, None])
    emb = jnp.concatenate(embs, axis=1)
    d = depth_ref[...]
    dm = jnp.mean(d)
    dstd = jnp.sqrt(jnp.mean((d - dm) ** 2))
    dn = (d - dm) / (dstd + 1e-6)
    z1 = (jnp.dot(emb, w1a_ref[...], preferred_element_type=jnp.float32)
          + dn * w1b_ref[...] + b1_ref[...])
    h1 = jnp.maximum(z1, 0.0)
    out_ref[...] = (jnp.dot(h1, w2_ref[...], preferred_element_type=jnp.float32)
                    + b2_ref[...])


def _final(agg, root, bias, batch, depth, w1a, w1b, b1, w2, b2):
    return pl.pallas_call(
        _tc_final_body,
        out_shape=jax.ShapeDtypeStruct((B, 1), jnp.float32),
    )(agg, root, bias, batch, depth, w1a, w1b, b1, w2, b2)


def kernel(state_x, state_edge_index, state_edge_type, state_batch,
           goal_x, goal_edge_index, goal_edge_type, goal_batch, depth,
           s1_W, s1_root, s1_bias, s2_W, s2_root, s2_bias,
           g1_W, g1_root, g1_bias, g2_W, g2_root, g2_bias,
           reg_w1, reg_b1, reg_w2, reg_b2):
    x = jnp.stack([state_x, goal_x])
    w1 = jnp.stack([s1_W, g1_W])
    rt1 = jnp.stack([s1_root, g1_root])
    b1 = jnp.stack([s1_bias, g1_bias]).reshape(NC, 1, H)
    w2 = jnp.stack([s2_W, g2_W])
    rt2 = jnp.stack([s2_root, g2_root])
    b2 = jnp.stack([s2_bias, g2_bias]).reshape(NC, 1, H)

    src = jnp.stack([state_edge_index[0], goal_edge_index[0]])
    dst = jnp.stack([state_edge_index[1], goal_edge_index[1]])
    et = jnp.stack([state_edge_type, goal_edge_type])
    seg = (dst * R + et).reshape(NC, NS, NCHUNK, CH)
    enc_off = jnp.arange(NC, dtype=jnp.int32)[:, None] * (R * N)
    gidx = (2 * (et * NP + jnp.remainder(src, NP)) + src // NP
            + enc_off).reshape(NC, NS, 1, ET)
    dst4 = dst.reshape(NC, NS, NCHUNK, CH)
    batch = jnp.stack([state_batch, goal_batch])

    t1, root1 = _transform(x, w1, rt1, D)
    w_edge, agg1 = _sc_weights_agg1(seg, t1.reshape(NC * R * N, H), gidx, dst4)
    t2, root2 = _layer2(agg1.reshape(NC, N, H), root1, b1, w2, rt2)
    agg2 = _sc_edge_agg(t2.reshape(NC * R * N, H), gidx, w_edge, dst4)
    agg2 = agg2.reshape(NC, N, H)

    pred = _final(agg2, root2, b2, batch, depth.reshape(B, 1),
                  reg_w1[:2 * H], reg_w1[2 * H:], reg_b1.reshape(1, H),
                  reg_w2, reg_b2.reshape(1, 1))
    return pred[:, 0]


# speedup vs baseline: 7.8728x; 1.0808x over previous
import functools

import jax
import jax.numpy as jnp
from jax import lax
from jax.experimental import pallas as pl
from jax.experimental.pallas import tpu as pltpu
from jax.experimental.pallas import tpu_sc as plsc

N = 10000
E = 160000
D = 128
H = 64
R = 32
B = 64

NC = 2
NS = 16
ET = E // NS
CH = 80
NCHUNK = ET // CH
NROW = N // NS
NRCNT = (N * R) // NS
NP = N // 2

_MESH = plsc.VectorSubcoreMesh(core_axis_name="c", subcore_axis_name="s")
_SC_PARAMS = pltpu.CompilerParams(use_tc_tiling_on_sc=False)

_EDGE_AGG_SCRATCH = [
    pltpu.VMEM((ET,), jnp.int32),
    pltpu.VMEM((ET,), jnp.float32),
    pltpu.VMEM((NCHUNK, CH), jnp.int32),
    pltpu.VMEM((CH, H), jnp.float32),
    pltpu.VMEM((CH, H), jnp.float32),
    pltpu.VMEM((CH, H), jnp.float32),
    pltpu.VMEM((CH, H), jnp.float32),
    pltpu.VMEM((NROW // 5, H), jnp.float32),
    pltpu.VMEM_SHARED((N, H), jnp.float32),
    pltpu.SemaphoreType.DMA,
    pltpu.SemaphoreType.DMA,
    pltpu.SemaphoreType.DMA,
    pltpu.SemaphoreType.DMA,
]


def _zero_fill(ref, nrows):
    def body(i, carry):
        for k in range(H // 16):
            ref[i, pl.ds(k * 16, 16)] = jnp.zeros((16,), jnp.float32)
        return carry

    lax.fori_loop(0, nrows, body, 0)


def _edge_agg_pipeline(t_hbm, agg_hbm, gix_v, w_v, dst_v,
                       g0_v, g1_v, s0_v, s1_v, zb_v, agg_sh,
                       semg0, semg1, sems0, sems1, c, s):
    _zero_fill(zb_v, NROW // 5)
    _zero_fill(s0_v, CH)
    _zero_fill(s1_v, CH)
    for p in range(5):
        pltpu.sync_copy(zb_v, agg_sh.at[pl.ds(s * NROW + p * (NROW // 5),
                                              NROW // 5)])
    plsc.subcore_barrier()

    def _gather(j, buf, sem):
        return pltpu.async_copy(t_hbm.at[gix_v.at[pl.ds(j * CH, CH)]], buf, sem)

    def _scale(j_base, gbuf, sbuf):
        for g in range(CH // 16):
            wv16 = w_v[pl.ds(j_base + g * 16, 16)]
            for i in range(16):
                e = g * 16 + i
                wb = jnp.full((16,), wv16[i], jnp.float32)
                for k in range(H // 16):
                    sl = pl.ds(k * 16, 16)
                    sbuf[e, sl] = gbuf[e, sl] * wb

    def _scatter(j, buf, sem):
        return pltpu.async_copy(buf, agg_sh.at[dst_v.at[j]], sem, add=True)

    _gather(0, g0_v, semg0)
    _gather(1, g1_v, semg1)
    _scatter(0, s0_v, sems0)
    _scatter(0, s1_v, sems1)

    def chunk_pair(m, carry):
        j0 = 2 * m
        j1 = j0 + 1
        pltpu.make_async_copy(t_hbm, g0_v, semg0).wait()
        pltpu.make_async_copy(s0_v, agg_sh.at[dst_v.at[0]], sems0).wait()
        _scale(j0 * CH, g0_v, s0_v)
        _scatter(j0, s0_v, sems0)
        _gather(j0 + 2, g0_v, semg0)
        pltpu.make_async_copy(t_hbm, g1_v, semg1).wait()
        pltpu.make_async_copy(s1_v, agg_sh.at[dst_v.at[0]], sems1).wait()
        _scale(j1 * CH, g1_v, s1_v)
        _scatter(j1, s1_v, sems1)
        _gather(jnp.minimum(j1 + 2, NCHUNK - 1), g1_v, semg1)
        return carry

    lax.fori_loop(0, (NCHUNK - 1) // 2, chunk_pair, 0)
    pltpu.make_async_copy(t_hbm, g0_v, semg0).wait()
    pltpu.make_async_copy(s0_v, agg_sh.at[dst_v.at[0]], sems0).wait()
    _scale((NCHUNK - 1) * CH, g0_v, s0_v)
    _scatter(NCHUNK - 1, s0_v, sems0)
    pltpu.make_async_copy(t_hbm, g1_v, semg1).wait()
    pltpu.make_async_copy(s0_v, agg_sh.at[dst_v.at[0]], sems0).wait()
    pltpu.make_async_copy(s1_v, agg_sh.at[dst_v.at[0]], sems1).wait()
    plsc.subcore_barrier()
    pltpu.sync_copy(agg_sh.at[pl.ds(s * NROW, NROW)], agg_hbm.at[c, s])


@functools.partial(
    pl.kernel,
    out_type=jax.ShapeDtypeStruct((NC, NS, 1, ET), jnp.float32),
    mesh=_MESH,
    scratch_types=[
        pltpu.VMEM((NCHUNK, CH), jnp.int32),
        pltpu.VMEM((CH,), jnp.float32),
        pltpu.VMEM((ET,), jnp.float32),
        pltpu.VMEM_SHARED((N * R,), jnp.float32),
    ],
    compiler_params=_SC_PARAMS,
)
def _sc_weights(seg_hbm, w_hbm, seg_v, ones_v, cbuf_v, cnt_sh):
    c = lax.axis_index("c")
    s = lax.axis_index("s")
    pltpu.sync_copy(seg_hbm.at[c, s], seg_v)
    for k in range(CH // 16):
        ones_v[pl.ds(k * 16, 16)] = jnp.ones((16,), jnp.float32)

    def zero_body(i, carry):
        cbuf_v[pl.ds(i * 16, 16)] = jnp.zeros((16,), jnp.float32)
        return carry

    lax.fori_loop(0, ET // 16, zero_body, 0)
    pltpu.sync_copy(cbuf_v, cnt_sh.at[pl.ds(s * NRCNT, ET)])
    pltpu.sync_copy(cbuf_v, cnt_sh.at[pl.ds(s * NRCNT + ET, ET)])
    plsc.subcore_barrier()

    def hist_body(j, carry):
        pltpu.sync_copy(ones_v, cnt_sh.at[seg_v.at[j]], add=True)
        return carry

    lax.fori_loop(0, NCHUNK, hist_body, 0)
    plsc.subcore_barrier()

    def gather_body(j, carry):
        pltpu.sync_copy(cnt_sh.at[seg_v.at[j]], cbuf_v.at[pl.ds(j * CH, CH)])
        return carry

    lax.fori_loop(0, NCHUNK, gather_body, 0)

    def w_body(i, carry):
        cv = cbuf_v[pl.ds(i * 16, 16)]
        cbuf_v[pl.ds(i * 16, 16)] = 1.0 / jnp.maximum(cv, 1.0)
        return carry

    lax.fori_loop(0, ET // 16, w_body, 0)
    pltpu.sync_copy(cbuf_v, w_hbm.at[c, s, 0])


@functools.partial(
    pl.kernel,
    out_type=jax.ShapeDtypeStruct((NC, NS, NROW, H), jnp.float32),
    mesh=_MESH,
    scratch_types=_EDGE_AGG_SCRATCH,
    compiler_params=_SC_PARAMS,
)
def _sc_edge_agg(t_hbm, gidx_hbm, w_hbm, dst_hbm, agg_hbm,
                 gix_v, w_v, dst_v, g0_v, g1_v, s0_v, s1_v, zb_v, agg_sh,
                 semg0, semg1, sems0, sems1):
    c = lax.axis_index("c")
    s = lax.axis_index("s")
    pltpu.sync_copy(gidx_hbm.at[c, s, 0], gix_v)
    pltpu.sync_copy(w_hbm.at[c, s, 0], w_v)
    pltpu.sync_copy(dst_hbm.at[c, s], dst_v)
    _edge_agg_pipeline(t_hbm, agg_hbm, gix_v, w_v, dst_v,
                       g0_v, g1_v, s0_v, s1_v, zb_v, agg_sh,
                       semg0, semg1, sems0, sems1, c, s)


def _tc_transform_body(x_ref, w_ref, rt_ref, t_ref, root_ref):
    r = pl.program_id(1)
    x = x_ref[0]
    w = w_ref[0, 0]
    t_ref[0, 0, :, 0:H] = jnp.dot(x[0:NP], w, preferred_element_type=jnp.float32)
    t_ref[0, 0, :, H:2 * H] = jnp.dot(x[NP:N], w,
                                      preferred_element_type=jnp.float32)

    @pl.when(r == 0)
    def _():
        root_ref[0] = jnp.dot(x, rt_ref[0], preferred_element_type=jnp.float32)


def _transform(x, w, rt, din):
    return pl.pallas_call(
        _tc_transform_body,
        grid=(NC, R),
        in_specs=[
            pl.BlockSpec((1, N, din), lambda s, r: (s, 0, 0)),
            pl.BlockSpec((1, 1, din, H), lambda s, r: (s, r, 0, 0)),
            pl.BlockSpec((1, din, H), lambda s, r: (s, 0, 0)),
        ],
        out_specs=[
            pl.BlockSpec((1, 1, NP, 2 * H), lambda s, r: (s, r, 0, 0)),
            pl.BlockSpec((1, N, H), lambda s, r: (s, 0, 0)),
        ],
        out_shape=[
            jax.ShapeDtypeStruct((NC, R, NP, 2 * H), jnp.float32),
            jax.ShapeDtypeStruct((NC, N, H), jnp.float32),
        ],
    )(x, w, rt)


def _tc_layer2_body(agg_ref, root_ref, bias_ref, w_ref, rt_ref, t_ref, root2_ref):
    r = pl.program_id(1)
    h = jnp.maximum(agg_ref[0] + root_ref[0] + bias_ref[0], 0.0)
    w = w_ref[0, 0]
    t_ref[0, 0, :, 0:H] = jnp.dot(h[0:NP], w, preferred_element_type=jnp.float32)
    t_ref[0, 0, :, H:2 * H] = jnp.dot(h[NP:N], w,
                                      preferred_element_type=jnp.float32)

    @pl.when(r == 0)
    def _():
        root2_ref[0] = jnp.dot(h, rt_ref[0], preferred_element_type=jnp.float32)


def _layer2(agg, root, bias, w, rt):
    return pl.pallas_call(
        _tc_layer2_body,
        grid=(NC, R),
        in_specs=[
            pl.BlockSpec((1, N, H), lambda s, r: (s, 0, 0)),
            pl.BlockSpec((1, N, H), lambda s, r: (s, 0, 0)),
            pl.BlockSpec((1, 1, H), lambda s, r: (s, 0, 0)),
            pl.BlockSpec((1, 1, H, H), lambda s, r: (s, r, 0, 0)),
            pl.BlockSpec((1, H, H), lambda s, r: (s, 0, 0)),
        ],
        out_specs=[
            pl.BlockSpec((1, 1, NP, 2 * H), lambda s, r: (s, r, 0, 0)),
            pl.BlockSpec((1, N, H), lambda s, r: (s, 0, 0)),
        ],
        out_shape=[
            jax.ShapeDtypeStruct((NC, R, NP, 2 * H), jnp.float32),
            jax.ShapeDtypeStruct((NC, N, H), jnp.float32),
        ],
    )(agg, root, bias, w, rt)


def _tc_final_body(agg_ref, root_ref, bias_ref, batch_ref, depth_ref,
                   w1a_ref, w1b_ref, b1_ref, w2_ref, b2_ref, out_ref):
    embs = []
    for s in range(2):
        h = jnp.maximum(agg_ref[s] + root_ref[s] + bias_ref[s], 0.0)
        bids = batch_ref[pl.ds(s, 1)]
        onehot = (lax.broadcasted_iota(jnp.int32, (B, N), 0) == bids
                  ).astype(jnp.float32)
        ssum = jnp.dot(onehot, h, preferred_element_type=jnp.float32)
        cnt = jnp.sum(onehot, axis=1)
        embs.append(ssum / jnp.maximum(cnt, 1.0)[:, None])
    emb = jnp.concatenate(embs, axis=1)
    d = depth_ref[...]
    dm = jnp.mean(d)
    dstd = jnp.sqrt(jnp.mean((d - dm) ** 2))
    dn = (d - dm) / (dstd + 1e-6)
    z1 = (jnp.dot(emb, w1a_ref[...], preferred_element_type=jnp.float32)
          + dn * w1b_ref[...] + b1_ref[...])
    h1 = jnp.maximum(z1, 0.0)
    out_ref[...] = (jnp.dot(h1, w2_ref[...], preferred_element_type=jnp.float32)
                    + b2_ref[...])


def _final(agg, root, bias, batch, depth, w1a, w1b, b1, w2, b2):
    return pl.pallas_call(
        _tc_final_body,
        out_shape=jax.ShapeDtypeStruct((B, 1), jnp.float32),
    )(agg, root, bias, batch, depth, w1a, w1b, b1, w2, b2)


def kernel(state_x, state_edge_index, state_edge_type, state_batch,
           goal_x, goal_edge_index, goal_edge_type, goal_batch, depth,
           s1_W, s1_root, s1_bias, s2_W, s2_root, s2_bias,
           g1_W, g1_root, g1_bias, g2_W, g2_root, g2_bias,
           reg_w1, reg_b1, reg_w2, reg_b2):
    x = jnp.stack([state_x, goal_x])
    w1 = jnp.stack([s1_W, g1_W])
    rt1 = jnp.stack([s1_root, g1_root])
    b1 = jnp.stack([s1_bias, g1_bias]).reshape(NC, 1, H)
    w2 = jnp.stack([s2_W, g2_W])
    rt2 = jnp.stack([s2_root, g2_root])
    b2 = jnp.stack([s2_bias, g2_bias]).reshape(NC, 1, H)

    src = jnp.stack([state_edge_index[0], goal_edge_index[0]])
    dst = jnp.stack([state_edge_index[1], goal_edge_index[1]])
    et = jnp.stack([state_edge_type, goal_edge_type])
    seg = (dst * R + et).reshape(NC, NS, NCHUNK, CH)
    enc_off = jnp.arange(NC, dtype=jnp.int32)[:, None] * (R * N)
    gidx = (2 * (et * NP + jnp.remainder(src, NP)) + src // NP
            + enc_off).reshape(NC, NS, 1, ET)
    dst4 = dst.reshape(NC, NS, NCHUNK, CH)
    batch = jnp.stack([state_batch, goal_batch])

    w_edge = _sc_weights(seg)
    t1, root1 = _transform(x, w1, rt1, D)
    agg1 = _sc_edge_agg(t1.reshape(NC * R * N, H), gidx, w_edge, dst4)
    t2, root2 = _layer2(agg1.reshape(NC, N, H), root1, b1, w2, rt2)
    agg2 = _sc_edge_agg(t2.reshape(NC * R * N, H), gidx, w_edge, dst4)
    agg2 = agg2.reshape(NC, N, H)

    pred = _final(agg2, root2, b2, batch, depth.reshape(B, 1),
                  reg_w1[:2 * H], reg_w1[2 * H:], reg_b1.reshape(1, H),
                  reg_w2, reg_b2.reshape(1, 1))
    return pred[:, 0]
